# baseline (speedup 1.0000x reference)
import jax
import jax.numpy as jnp
from jax import lax
from jax.experimental import pallas as pl
from jax.experimental.pallas import tpu as pltpu
from jax.experimental.pallas import tpu_sc as plsc

BATCH = 1024
MAXLEN = 200
N_FEAT = 64
HALF = MAXLEN // 2


def _emb_body(x_hbm, pe_hbm, E_hbm, out_hbm, idx_v, rows_v, pe_v, sem):
    info = plsc.get_sparse_core_info()
    nc, ns = info.num_cores, info.num_subcores
    nw = nc * ns
    wid = lax.axis_index("s") * nc + lax.axis_index("c")
    rows_per_w = BATCH // nw

    pltpu.sync_copy(pe_hbm, pe_v)

    def per_row(b, carry):
        row = wid * rows_per_w + b
        pltpu.sync_copy(x_hbm.at[row], idx_v)
        cp0 = pltpu.async_copy(
            E_hbm.at[idx_v.at[0]], rows_v.at[pl.ds(0, HALF)], sem)
        cp1 = pltpu.async_copy(
            E_hbm.at[idx_v.at[1]], rows_v.at[pl.ds(HALF, HALF)], sem)
        cp0.wait()
        cp1.wait()

        def add_pe(r, c2):
            for c in range(N_FEAT // 16):
                sl = pl.ds(c * 16, 16)
                rows_v[r, sl] = rows_v[r, sl] + pe_v[r, sl]
            return c2

        lax.fori_loop(0, MAXLEN, add_pe, 0)
        pltpu.sync_copy(rows_v, out_hbm.at[row])
        return carry

    lax.fori_loop(0, rows_per_w, per_row, 0)


def kernel(x, E, pe):
    pe2 = pe.reshape(MAXLEN, N_FEAT)
    x3 = x.reshape(BATCH, 2, HALF)
    mesh = plsc.VectorSubcoreMesh(core_axis_name="c", subcore_axis_name="s")
    f = pl.kernel(
        _emb_body,
        out_type=jax.ShapeDtypeStruct((BATCH, MAXLEN, N_FEAT), jnp.float32),
        mesh=mesh,
        compiler_params=pltpu.CompilerParams(use_tc_tiling_on_sc=False),
        scratch_types=[
            pltpu.VMEM((2, HALF), jnp.int32),
            pltpu.VMEM((MAXLEN, N_FEAT), jnp.float32),
            pltpu.VMEM((MAXLEN, N_FEAT), jnp.float32),
            pltpu.SemaphoreType.DMA,
        ],
    )
    return f(x3, pe2, E)


# speedup vs baseline: 1.0627x; 1.0627x over previous
import jax
import jax.numpy as jnp
from jax import lax
from jax.experimental import pallas as pl
from jax.experimental.pallas import tpu as pltpu
from jax.experimental.pallas import tpu_sc as plsc

BATCH = 1024
MAXLEN = 200
N_FEAT = 64
HALF = MAXLEN // 2
NBUF = 4


def _emb_body(x_hbm, pe_hbm, E_hbm, out_hbm, idx_v, rows_v, pe_v,
              sems_g, sems_w):
    info = plsc.get_sparse_core_info()
    nc, ns = info.num_cores, info.num_subcores
    nw = nc * ns
    wid = lax.axis_index("s") * nc + lax.axis_index("c")
    rows_per_w = BATCH // nw
    base = wid * rows_per_w

    pltpu.sync_copy(pe_hbm, pe_v)

    def gather_copies(b, u):
        idx_b, rows_b, sem = idx_v.at[u], rows_v.at[u], sems_g[u]
        return (
            pltpu.make_async_copy(
                E_hbm.at[idx_b.at[0]], rows_b.at[pl.ds(0, HALF)], sem),
            pltpu.make_async_copy(
                E_hbm.at[idx_b.at[1]], rows_b.at[pl.ds(HALF, HALF)], sem),
        )

    def issue_gather(b, u):
        pltpu.sync_copy(x_hbm.at[base + b], idx_v.at[u])
        for cp in gather_copies(b, u):
            cp.start()

    def wait_gather(b, u):
        for cp in gather_copies(b, u):
            cp.wait()

    def wb_copy(b, u):
        return pltpu.make_async_copy(rows_v.at[u], out_hbm.at[base + b],
                                     sems_w[u])

    issue_gather(0, 0)

    def group(g, carry):
        for u in range(NBUF):
            b = g * NBUF + u
            nxt = (u + 1) % NBUF

            @pl.when(b + 1 < rows_per_w)
            def _():
                @pl.when(b >= NBUF - 1)
                def _():
                    wb_copy(b + 1 - NBUF, nxt).wait()
                issue_gather(b + 1, nxt)

            wait_gather(b, u)

            rows_b = rows_v.at[u]

            @plsc.parallel_loop(0, MAXLEN, step=1, unroll=4)
            def _(r):
                for c in range(N_FEAT // 16):
                    sl = pl.ds(c * 16, 16)
                    rows_b[r, sl] = rows_b[r, sl] + pe_v[r, sl]

            wb_copy(b, u).start()
        return carry

    lax.fori_loop(0, rows_per_w // NBUF, group, 0)

    for u in range(NBUF):
        b = rows_per_w - NBUF + u
        wb_copy(b, b % NBUF).wait()


def kernel(x, E, pe):
    pe2 = pe.reshape(MAXLEN, N_FEAT)
    x3 = x.reshape(BATCH, 2, HALF)
    mesh = plsc.VectorSubcoreMesh(core_axis_name="c", subcore_axis_name="s")
    f = pl.kernel(
        _emb_body,
        out_type=jax.ShapeDtypeStruct((BATCH, MAXLEN, N_FEAT), jnp.float32),
        mesh=mesh,
        compiler_params=pltpu.CompilerParams(use_tc_tiling_on_sc=False),
        scratch_types=[
            pltpu.VMEM((NBUF, 2, HALF), jnp.int32),
            pltpu.VMEM((NBUF, MAXLEN, N_FEAT), jnp.float32),
            pltpu.VMEM((MAXLEN, N_FEAT), jnp.float32),
            [pltpu.SemaphoreType.DMA] * NBUF,
            [pltpu.SemaphoreType.DMA] * NBUF,
        ],
    )
    return f(x3, pe2, E)


# speedup vs baseline: 1.0807x; 1.0169x over previous
import jax
import jax.numpy as jnp
from jax import lax
from jax.experimental import pallas as pl
from jax.experimental.pallas import tpu as pltpu
from jax.experimental.pallas import tpu_sc as plsc

BATCH = 1024
MAXLEN = 200
N_FEAT = 64
CHUNK = 128
N_FLAT = BATCH * MAXLEN
N_CHUNKS = N_FLAT // CHUNK
NBUF = 10
DEPTH = 8


def _emb_body(x_hbm, pe_hbm, E_hbm, out_hbm, idx_v, rows_v, pe_v,
              sems_g, sems_w):
    info = plsc.get_sparse_core_info()
    nc, ns = info.num_cores, info.num_subcores
    nw = nc * ns
    wid = lax.axis_index("s") * nc + lax.axis_index("c")
    chunks_per_w = N_CHUNKS // nw
    cbase = wid * chunks_per_w

    pltpu.sync_copy(pe_hbm, pe_v)
    pltpu.sync_copy(x_hbm.at[pl.ds(cbase, chunks_per_w)], idx_v)

    def g_copy(c, u):
        return pltpu.make_async_copy(
            E_hbm.at[idx_v.at[c]], rows_v.at[u], sems_g[u])

    def w_copy(c, u):
        return pltpu.make_async_copy(
            rows_v.at[u], out_hbm.at[cbase + c], sems_w[u])

    def g_start(c, u):
        pltpu.async_copy(E_hbm.at[idx_v.at[c]], rows_v.at[u], sems_g[u])

    for d in range(DEPTH):
        g_start(d, d)

    def group(g, carry):
        for u in range(NBUF):
            c = g * NBUF + u
            nxt = (u + DEPTH) % NBUF

            @pl.when(c + DEPTH < chunks_per_w)
            def _():
                @pl.when(c >= NBUF - DEPTH)
                def _():
                    w_copy(c + DEPTH - NBUF, nxt).wait()
                g_start(c + DEPTH, nxt)

            g_copy(c, u).wait()

            rows_b = rows_v.at[u]
            t0 = lax.rem(c * CHUNK, MAXLEN)

            @plsc.parallel_loop(0, CHUNK, step=1, unroll=4)
            def _(r):
                t = t0 + r
                t = jnp.where(t >= MAXLEN, t - MAXLEN, t)
                for cc in range(N_FEAT // 16):
                    sl = pl.ds(cc * 16, 16)
                    rows_b[r, sl] = rows_b[r, sl] + pe_v[t, sl]

            w_copy(c, u).start()
        return carry

    lax.fori_loop(0, chunks_per_w // NBUF, group, 0)

    for u in range(NBUF):
        c = chunks_per_w - NBUF + u
        w_copy(c, c % NBUF).wait()


def kernel(x, E, pe):
    pe2 = pe.reshape(MAXLEN, N_FEAT)
    x3 = x.reshape(N_CHUNKS, CHUNK)
    mesh = plsc.VectorSubcoreMesh(core_axis_name="c", subcore_axis_name="s")
    f = pl.kernel(
        _emb_body,
        out_type=jax.ShapeDtypeStruct((N_CHUNKS, CHUNK, N_FEAT), jnp.float32),
        mesh=mesh,
        compiler_params=pltpu.CompilerParams(use_tc_tiling_on_sc=False),
        scratch_types=[
            pltpu.VMEM((N_CHUNKS // 32, CHUNK), jnp.int32),
            pltpu.VMEM((NBUF, CHUNK, N_FEAT), jnp.float32),
            pltpu.VMEM((MAXLEN, N_FEAT), jnp.float32),
            [pltpu.SemaphoreType.DMA] * NBUF,
            [pltpu.SemaphoreType.DMA] * NBUF,
        ],
    )
    out = f(x3, pe2, E)
    return out.reshape(BATCH, MAXLEN, N_FEAT)
